# Initial kernel scaffold; baseline (speedup 1.0000x reference)
import jax
import jax.numpy as jnp
from jax.experimental import pallas as pl


def kernel(x, edge_index, edge_weight, W1, root1, b1, W2, root2, b2):
    raise NotImplementedError("write your pallas kernel here")


# speedup vs baseline: 2.9822x; 2.9822x over previous
import jax
import jax.numpy as jnp
from jax import lax
from jax.experimental import pallas as pl
from jax.experimental.pallas import tpu as pltpu
from jax.experimental.pallas import tpu_sc as plsc

N = 10000
E = 320000
F = 128
NC = 2
NS = 16
L = 16

HALF = N // NC
STRIPE = 320
ACC_ROWS = NS * STRIPE + 8
TRASH = NS * STRIPE
EPT = E // NS
CH = 80
NCHUNK = EPT // CH

_MESH = plsc.VectorSubcoreMesh(core_axis_name="c", subcore_axis_name="s")


def _sc_pass_body(feat, srcl, dstl, ul, gall_o, gu_o,
                  rows, srows, ch_src, ch_d0, ch_dst, ch_u,
                  acc_all, acc_u, sem):
    c = lax.axis_index("c")
    s = lax.axis_index("s")
    lo = c * HALF

    def _init_row(i, _):
        for v in range(F // L):
            rows[i, pl.ds(v * L, L)] = jnp.zeros((L,), jnp.float32)
        return 0
    lax.fori_loop(0, CH, _init_row, 0)

    for j in range(STRIPE // CH):
        off = s * STRIPE + j * CH
        pltpu.sync_copy(rows, acc_all.at[pl.ds(off, CH)])
        pltpu.sync_copy(rows, acc_u.at[pl.ds(off, CH)])
    plsc.subcore_barrier()

    base_e = s * EPT

    def _chunk(j, _):
        b = base_e + j * CH
        pltpu.sync_copy(srcl.at[pl.ds(b, CH)], ch_src)
        pltpu.sync_copy(dstl.at[pl.ds(b, CH)], ch_d0)
        pltpu.sync_copy(ul.at[pl.ds(b, CH)], ch_u)
        for v in range(CH // L):
            dv = ch_d0[pl.ds(v * L, L)]
            dl = dv - lo
            mask = (dl >= 0) & (dl < HALF)
            ch_dst[pl.ds(v * L, L)] = jnp.where(mask, dl, TRASH)
        pltpu.async_copy(feat.at[ch_src], rows, sem).wait()
        for g in range(CH // L):
            ub = ch_u[pl.ds(g * L, L)]
            for rr in range(L):
                r = g * L + rr
                ubc = lax.gather(
                    ub, jnp.full((L, 1), rr, jnp.int32),
                    lax.GatherDimensionNumbers((), (0,), (0,)), (1,),
                    mode=lax.GatherScatterMode.PROMISE_IN_BOUNDS)
                for v in range(F // L):
                    srows[r, pl.ds(v * L, L)] = rows[r, pl.ds(v * L, L)] * ubc
        pltpu.sync_copy(rows, acc_all.at[ch_dst], add=True)
        pltpu.sync_copy(srows, acc_u.at[ch_dst], add=True)
        return 0
    lax.fori_loop(0, NCHUNK, _chunk, 0)

    plsc.subcore_barrier()

    for j in range(STRIPE // CH):
        off = s * STRIPE + j * CH
        pltpu.sync_copy(acc_all.at[pl.ds(off, CH)], rows)
        pltpu.sync_copy(rows, gall_o.at[c, pl.ds(off, CH)])
        pltpu.sync_copy(acc_u.at[pl.ds(off, CH)], srows)
        pltpu.sync_copy(srows, gu_o.at[c, pl.ds(off, CH)])


_sc_pass = pl.kernel(
    _sc_pass_body,
    out_type=(
        jax.ShapeDtypeStruct((NC, NS * STRIPE, F), jnp.float32),
        jax.ShapeDtypeStruct((NC, NS * STRIPE, F), jnp.float32),
    ),
    mesh=_MESH,
    scratch_types=[
        pltpu.VMEM((CH, F), jnp.float32),
        pltpu.VMEM((CH, F), jnp.float32),
        pltpu.VMEM((CH,), jnp.int32),
        pltpu.VMEM((CH,), jnp.int32),
        pltpu.VMEM((CH,), jnp.int32),
        pltpu.VMEM((CH,), jnp.float32),
        pltpu.VMEM_SHARED((ACC_ROWS, F), jnp.float32),
        pltpu.VMEM_SHARED((ACC_ROWS, F), jnp.float32),
        pltpu.SemaphoreType.DMA,
    ],
)


def _sc_cnt_body(dstl, cnt_o, ones_b, ch_d0, ch_dst, acc_cnt):
    c = lax.axis_index("c")
    s = lax.axis_index("s")
    lo = c * HALF

    def _init_row(i, _):
        for v in range(F // L):
            ones_b[i, pl.ds(v * L, L)] = jnp.zeros((L,), jnp.float32)
        return 0
    lax.fori_loop(0, CH, _init_row, 0)
    for j in range(STRIPE // CH):
        off = s * STRIPE + j * CH
        pltpu.sync_copy(ones_b, acc_cnt.at[pl.ds(off, CH)])
    plsc.subcore_barrier()

    def _one_row(i, _):
        for v in range(F // L):
            ones_b[i, pl.ds(v * L, L)] = jnp.ones((L,), jnp.float32)
        return 0
    lax.fori_loop(0, CH, _one_row, 0)

    base_e = s * EPT

    def _chunk(j, _):
        b = base_e + j * CH
        pltpu.sync_copy(dstl.at[pl.ds(b, CH)], ch_d0)
        for v in range(CH // L):
            dv = ch_d0[pl.ds(v * L, L)]
            dl = dv - lo
            mask = (dl >= 0) & (dl < HALF)
            ch_dst[pl.ds(v * L, L)] = jnp.where(mask, dl, TRASH)
        pltpu.sync_copy(ones_b, acc_cnt.at[ch_dst], add=True)
        return 0
    lax.fori_loop(0, NCHUNK, _chunk, 0)

    plsc.subcore_barrier()

    for j in range(STRIPE // CH):
        off = s * STRIPE + j * CH
        pltpu.sync_copy(acc_cnt.at[pl.ds(off, CH)], ones_b)
        pltpu.sync_copy(ones_b, cnt_o.at[c, pl.ds(off, CH)])


_sc_cnt = pl.kernel(
    _sc_cnt_body,
    out_type=(jax.ShapeDtypeStruct((NC, NS * STRIPE, F), jnp.float32),),
    mesh=_MESH,
    scratch_types=[
        pltpu.VMEM((CH, F), jnp.float32),
        pltpu.VMEM((CH,), jnp.int32),
        pltpu.VMEM((CH,), jnp.int32),
        pltpu.VMEM_SHARED((ACC_ROWS, F), jnp.float32),
    ],
)


def _dense1_body(gall, gu, cnt, x, w0, w1, root, b, o):
    ga = jnp.concatenate([gall[0, :HALF], gall[1, :HALF]], axis=0)
    gm = jnp.concatenate([gu[0, :HALF], gu[1, :HALF]], axis=0)
    cn = jnp.concatenate([cnt[0, :HALF, 0:1], cnt[1, :HALF, 0:1]], axis=0)
    agg = jnp.dot(ga, w0[...], preferred_element_type=jnp.float32)
    agg = agg + jnp.dot(gm, w1[...] - w0[...], preferred_element_type=jnp.float32)
    inv = 1.0 / jnp.maximum(cn, 1.0)
    v = agg * inv + jnp.dot(x[...], root[...], preferred_element_type=jnp.float32) + b[...]
    o[...] = jnp.where(v > 0, v, jnp.exp(jnp.minimum(v, 0.0)) - 1.0)


def _dense2_body(gall, gu, cnt, h, w0, w1, root, b, o):
    ga = jnp.concatenate([gall[0, :HALF], gall[1, :HALF]], axis=0)
    gm = jnp.concatenate([gu[0, :HALF], gu[1, :HALF]], axis=0)
    cn = jnp.concatenate([cnt[0, :HALF, 0:1], cnt[1, :HALF, 0:1]], axis=0)
    agg = jnp.dot(ga, w0[...], preferred_element_type=jnp.float32)
    agg = agg + jnp.dot(gm, w1[...] - w0[...], preferred_element_type=jnp.float32)
    inv = 1.0 / jnp.maximum(cn, 1.0)
    o[...] = agg * inv + jnp.dot(h[...], root[...], preferred_element_type=jnp.float32) + b[...]


def kernel(x, edge_index, edge_weight, W1, root1, b1, W2, root2, b2):
    src = edge_index[0]
    dst = edge_index[1]
    u = edge_weight.reshape(-1)

    (cnt,) = _sc_cnt(dst)
    gall1, gu1 = _sc_pass(x, src, dst, u)
    h = pl.pallas_call(
        _dense1_body,
        out_shape=jax.ShapeDtypeStruct((N, F), jnp.float32),
    )(gall1, gu1, cnt, x, W1[0], W1[1], root1, b1)

    gall2, gu2 = _sc_pass(h, src, dst, u)
    out = pl.pallas_call(
        _dense2_body,
        out_shape=jax.ShapeDtypeStruct((N, W2.shape[2]), jnp.float32),
    )(gall2, gu2, cnt, h, W2[0], W2[1], root2, b2)
    return out


# speedup vs baseline: 3.8504x; 1.2911x over previous
import jax
import jax.numpy as jnp
from jax import lax
from jax.experimental import pallas as pl
from jax.experimental.pallas import tpu as pltpu
from jax.experimental.pallas import tpu_sc as plsc

N = 10000
E = 320000
F = 128
NC = 2
NS = 16
L = 16

HALF = N // NC
STRIPE = 320
ACC_ROWS = NS * STRIPE + 8
TRASH = NS * STRIPE
EPT = E // NS
CH = 80
NCHUNK = EPT // CH

_MESH = plsc.VectorSubcoreMesh(core_axis_name="c", subcore_axis_name="s")


def _sc_pass_body(feat, srcl, dstl, ul, gall_o, gu_o,
                  rows, srows, rows1, ch_src, ch_d0, ch_dst, ch_u,
                  ch_src1, ch_d1, ch_dst1, ch_u1,
                  acc_all, acc_u, sem, sem1):
    c = lax.axis_index("c")
    s = lax.axis_index("s")
    lo = c * HALF

    def _init_row(i, _):
        for v in range(F // L):
            rows[i, pl.ds(v * L, L)] = jnp.zeros((L,), jnp.float32)
        return 0
    lax.fori_loop(0, CH, _init_row, 0)

    for j in range(STRIPE // CH):
        off = s * STRIPE + j * CH
        pltpu.sync_copy(rows, acc_all.at[pl.ds(off, CH)])
        pltpu.sync_copy(rows, acc_u.at[pl.ds(off, CH)])
    plsc.subcore_barrier()

    base_e = s * EPT

    def _mask(dsrc, ddst):
        for v in range(CH // L):
            dv = dsrc[pl.ds(v * L, L)]
            dl = dv - lo
            mask = (dl >= 0) & (dl < HALF)
            ddst[pl.ds(v * L, L)] = jnp.where(mask, dl, TRASH)

    def _scale(rbuf, ubuf, sbuf):
        for g in range(CH // L):
            ub = ubuf[pl.ds(g * L, L)]
            for rr in range(L):
                r = g * L + rr
                ubc = lax.gather(
                    ub, jnp.full((L, 1), rr, jnp.int32),
                    lax.GatherDimensionNumbers((), (0,), (0,)), (1,),
                    mode=lax.GatherScatterMode.PROMISE_IN_BOUNDS)
                for v in range(F // L):
                    sbuf[r, pl.ds(v * L, L)] = rbuf[r, pl.ds(v * L, L)] * ubc

    def _chunk2(j, _):
        b0 = base_e + (2 * j) * CH
        b1 = b0 + CH
        pltpu.sync_copy(srcl.at[pl.ds(b0, CH)], ch_src)
        pltpu.sync_copy(dstl.at[pl.ds(b0, CH)], ch_d0)
        pltpu.sync_copy(ul.at[pl.ds(b0, CH)], ch_u)
        g0 = pltpu.async_copy(feat.at[ch_src], rows, sem)
        pltpu.sync_copy(srcl.at[pl.ds(b1, CH)], ch_src1)
        pltpu.sync_copy(dstl.at[pl.ds(b1, CH)], ch_d1)
        pltpu.sync_copy(ul.at[pl.ds(b1, CH)], ch_u1)
        g1 = pltpu.async_copy(feat.at[ch_src1], rows1, sem1)
        _mask(ch_d0, ch_dst)
        _mask(ch_d1, ch_dst1)
        g0.wait()
        _scale(rows, ch_u, srows)
        pltpu.sync_copy(rows, acc_all.at[ch_dst], add=True)
        pltpu.sync_copy(srows, acc_u.at[ch_dst], add=True)
        g1.wait()
        _scale(rows1, ch_u1, srows)
        pltpu.sync_copy(rows1, acc_all.at[ch_dst1], add=True)
        pltpu.sync_copy(srows, acc_u.at[ch_dst1], add=True)
        return 0
    lax.fori_loop(0, NCHUNK // 2, _chunk2, 0)

    plsc.subcore_barrier()

    for j in range(STRIPE // CH):
        off = s * STRIPE + j * CH
        pltpu.sync_copy(acc_all.at[pl.ds(off, CH)], rows)
        pltpu.sync_copy(rows, gall_o.at[c, pl.ds(off, CH)])
        pltpu.sync_copy(acc_u.at[pl.ds(off, CH)], srows)
        pltpu.sync_copy(srows, gu_o.at[c, pl.ds(off, CH)])


_sc_pass = pl.kernel(
    _sc_pass_body,
    out_type=(
        jax.ShapeDtypeStruct((NC, NS * STRIPE, F), jnp.float32),
        jax.ShapeDtypeStruct((NC, NS * STRIPE, F), jnp.float32),
    ),
    mesh=_MESH,
    scratch_types=[
        pltpu.VMEM((CH, F), jnp.float32),
        pltpu.VMEM((CH, F), jnp.float32),
        pltpu.VMEM((CH, F), jnp.float32),
        pltpu.VMEM((CH,), jnp.int32),
        pltpu.VMEM((CH,), jnp.int32),
        pltpu.VMEM((CH,), jnp.int32),
        pltpu.VMEM((CH,), jnp.float32),
        pltpu.VMEM((CH,), jnp.int32),
        pltpu.VMEM((CH,), jnp.int32),
        pltpu.VMEM((CH,), jnp.int32),
        pltpu.VMEM((CH,), jnp.float32),
        pltpu.VMEM_SHARED((ACC_ROWS, F), jnp.float32),
        pltpu.VMEM_SHARED((ACC_ROWS, F), jnp.float32),
        pltpu.SemaphoreType.DMA,
        pltpu.SemaphoreType.DMA,
    ],
)


def _sc_cnt_body(dstl, cnt_o, ones_b, ch_d0, ch_dst, acc_cnt):
    c = lax.axis_index("c")
    s = lax.axis_index("s")
    lo = c * HALF

    def _init_row(i, _):
        for v in range(F // L):
            ones_b[i, pl.ds(v * L, L)] = jnp.zeros((L,), jnp.float32)
        return 0
    lax.fori_loop(0, CH, _init_row, 0)
    for j in range(STRIPE // CH):
        off = s * STRIPE + j * CH
        pltpu.sync_copy(ones_b, acc_cnt.at[pl.ds(off, CH)])
    plsc.subcore_barrier()

    def _one_row(i, _):
        for v in range(F // L):
            ones_b[i, pl.ds(v * L, L)] = jnp.ones((L,), jnp.float32)
        return 0
    lax.fori_loop(0, CH, _one_row, 0)

    base_e = s * EPT

    def _chunk(j, _):
        b = base_e + j * CH
        pltpu.sync_copy(dstl.at[pl.ds(b, CH)], ch_d0)
        for v in range(CH // L):
            dv = ch_d0[pl.ds(v * L, L)]
            dl = dv - lo
            mask = (dl >= 0) & (dl < HALF)
            ch_dst[pl.ds(v * L, L)] = jnp.where(mask, dl, TRASH)
        pltpu.sync_copy(ones_b, acc_cnt.at[ch_dst], add=True)
        return 0
    lax.fori_loop(0, NCHUNK, _chunk, 0)

    plsc.subcore_barrier()

    for j in range(STRIPE // CH):
        off = s * STRIPE + j * CH
        pltpu.sync_copy(acc_cnt.at[pl.ds(off, CH)], ones_b)
        pltpu.sync_copy(ones_b, cnt_o.at[c, pl.ds(off, CH)])


_sc_cnt = pl.kernel(
    _sc_cnt_body,
    out_type=(jax.ShapeDtypeStruct((NC, NS * STRIPE, F), jnp.float32),),
    mesh=_MESH,
    scratch_types=[
        pltpu.VMEM((CH, F), jnp.float32),
        pltpu.VMEM((CH,), jnp.int32),
        pltpu.VMEM((CH,), jnp.int32),
        pltpu.VMEM_SHARED((ACC_ROWS, F), jnp.float32),
    ],
)


def _dense1_body(gall, gu, cnt, x, w0, w1, root, b, o):
    ga = jnp.concatenate([gall[0, :HALF], gall[1, :HALF]], axis=0)
    gm = jnp.concatenate([gu[0, :HALF], gu[1, :HALF]], axis=0)
    cn = jnp.concatenate([cnt[0, :HALF, 0:1], cnt[1, :HALF, 0:1]], axis=0)
    agg = jnp.dot(ga, w0[...], preferred_element_type=jnp.float32)
    agg = agg + jnp.dot(gm, w1[...] - w0[...], preferred_element_type=jnp.float32)
    inv = 1.0 / jnp.maximum(cn, 1.0)
    v = agg * inv + jnp.dot(x[...], root[...], preferred_element_type=jnp.float32) + b[...]
    o[...] = jnp.where(v > 0, v, jnp.exp(jnp.minimum(v, 0.0)) - 1.0)


def _dense2_body(gall, gu, cnt, h, w0, w1, root, b, o):
    ga = jnp.concatenate([gall[0, :HALF], gall[1, :HALF]], axis=0)
    gm = jnp.concatenate([gu[0, :HALF], gu[1, :HALF]], axis=0)
    cn = jnp.concatenate([cnt[0, :HALF, 0:1], cnt[1, :HALF, 0:1]], axis=0)
    agg = jnp.dot(ga, w0[...], preferred_element_type=jnp.float32)
    agg = agg + jnp.dot(gm, w1[...] - w0[...], preferred_element_type=jnp.float32)
    inv = 1.0 / jnp.maximum(cn, 1.0)
    o[...] = agg * inv + jnp.dot(h[...], root[...], preferred_element_type=jnp.float32) + b[...]


def kernel(x, edge_index, edge_weight, W1, root1, b1, W2, root2, b2):
    src = edge_index[0]
    dst = edge_index[1]
    u = edge_weight.reshape(-1)

    (cnt,) = _sc_cnt(dst)
    gall1, gu1 = _sc_pass(x, src, dst, u)
    h = pl.pallas_call(
        _dense1_body,
        out_shape=jax.ShapeDtypeStruct((N, F), jnp.float32),
    )(gall1, gu1, cnt, x, W1[0], W1[1], root1, b1)

    gall2, gu2 = _sc_pass(h, src, dst, u)
    out = pl.pallas_call(
        _dense2_body,
        out_shape=jax.ShapeDtypeStruct((N, W2.shape[2]), jnp.float32),
    )(gall2, gu2, cnt, h, W2[0], W2[1], root2, b2)
    return out
